# baseline (speedup 1.0000x reference)
import jax
import jax.numpy as jnp
from jax.experimental import pallas as pl

B, T, D, H, DH, N, TOP_K = 1, 2048, 768, 12, 64, 1024, 8
SCALE = DH ** (-0.5)
NEG = float(jnp.finfo(jnp.float32).min)
T_BLK = 256


def _rope(x, cos, sin):
    half = DH // 2
    rot = jnp.concatenate([-x[:, half:], x[:, :half]], axis=1)
    return x * cos + rot * sin


def _head_kernel(hs_ref, hsb_ref, cos_ref, sin_ref, mem_ref, wq_ref, wk_ref,
                 wv_ref, ao_ref, om_ref):
    hs = hs_ref[...]
    cos = cos_ref[...]
    sin = sin_ref[...]
    q = _rope(jnp.dot(hs, wq_ref[...].T, preferred_element_type=jnp.float32),
              cos, sin)
    qh = q.astype(jnp.bfloat16)
    hsb = hsb_ref[...]
    kh = _rope(jnp.dot(hsb, wk_ref[...].astype(jnp.bfloat16).T,
                       preferred_element_type=jnp.float32),
               cos, sin).astype(jnp.bfloat16)
    v = jnp.dot(hsb, wv_ref[...].astype(jnp.bfloat16).T,
                preferred_element_type=jnp.float32).astype(jnp.bfloat16)
    for c in range(T // T_BLK):
        cols = (c + 1) * T_BLK
        qb = qh[c * T_BLK:(c + 1) * T_BLK, :]
        s = jnp.dot(qb, kh[:cols, :].T,
                    preferred_element_type=jnp.float32) * SCALE
        col = jax.lax.broadcasted_iota(jnp.int32, (T_BLK, cols), 1)
        row = jax.lax.broadcasted_iota(jnp.int32, (T_BLK, cols), 0)
        s = jnp.where(col <= row + c * T_BLK, s, NEG)
        e = jnp.exp(s)
        p = (e / jnp.sum(e, axis=1, keepdims=True)).astype(jnp.bfloat16)
        ao_ref[0, c * T_BLK:(c + 1) * T_BLK, :] = jnp.dot(
            p, v[:cols, :], preferred_element_type=jnp.float32).astype(
            jnp.bfloat16)
    mem = mem_ref[...]
    k_mem = jnp.dot(mem, wk_ref[...].T, preferred_element_type=jnp.float32)
    v_mem = jnp.dot(mem, wv_ref[...].T, preferred_element_type=jnp.float32)
    sim = jnp.dot(q, k_mem.T, preferred_element_type=jnp.float32) * SCALE
    p1 = jnp.full((T, N // 8), NEG, jnp.float32)
    p2 = p1
    p3 = p1
    for cnk in range(8):
        c = sim[:, cnk * (N // 8):(cnk + 1) * (N // 8)]
        lo = jnp.minimum(p1, c)
        p1 = jnp.maximum(p1, c)
        lo2 = jnp.minimum(p2, lo)
        p2 = jnp.maximum(p2, lo)
        p3 = jnp.maximum(p3, lo2)
    for it in range(TOP_K):
        m = jnp.max(p1, axis=1, keepdims=True)
        if it < TOP_K - 1:
            sel = p1 >= m
            p1 = jnp.where(sel, p2, p1)
            p2 = jnp.where(sel, p3, p2)
            p3 = jnp.where(sel, NEG, p3)
    w = jnp.where(sim >= m, jnp.exp(sim), 0.0)
    z = jnp.sum(w, axis=1, keepdims=True)
    om = jnp.dot(w.astype(jnp.bfloat16), v_mem.astype(jnp.bfloat16),
                 preferred_element_type=jnp.float32)
    om_ref[0] = (om / z).astype(jnp.bfloat16)


def _fusion_kernel(ao_ref, om_ref, wo_ref, f1_ref, f2_ref, b_ref, out_ref):
    ao = jnp.concatenate([ao_ref[h] for h in range(H)], axis=1)
    om = jnp.concatenate([om_ref[h] for h in range(H)], axis=1)
    o_local = jnp.dot(ao, wo_ref[...].astype(jnp.bfloat16).T,
                      preferred_element_type=jnp.float32)
    out = jnp.dot(o_local.astype(jnp.bfloat16),
                  f1_ref[...].astype(jnp.bfloat16).T,
                  preferred_element_type=jnp.float32)
    out += jnp.dot(om, f2_ref[...].astype(jnp.bfloat16).T,
                   preferred_element_type=jnp.float32)
    out_ref[...] = out + b_ref[...]


def kernel(hidden_states, cos, sin, memory, Wq, Wk, Wv, Wo, fusion_W, fusion_b):
    hs = hidden_states[0]
    cs = cos[0]
    sn = sin[0]
    mem = memory[0]
    f1 = fusion_W[:, :D]
    f2 = fusion_W[:, D:]

    head_w = pl.BlockSpec((DH, D), lambda h: (h, 0))
    full2d = lambda a, b: pl.BlockSpec((a, b), lambda h: (0, 0))
    out_head = pl.BlockSpec((1, T, DH), lambda h: (h, 0, 0))

    ao, om = pl.pallas_call(
        _head_kernel,
        grid=(H,),
        in_specs=[full2d(T, D), full2d(T, D), full2d(T, DH), full2d(T, DH),
                  full2d(N, D), head_w, head_w, head_w],
        out_specs=[out_head, out_head],
        out_shape=[jax.ShapeDtypeStruct((H, T, DH), jnp.bfloat16),
                   jax.ShapeDtypeStruct((H, T, DH), jnp.bfloat16)],
    )(hs, hs.astype(jnp.bfloat16), cs, sn, mem, Wq, Wk, Wv)

    out = pl.pallas_call(
        _fusion_kernel,
        in_specs=[
            pl.BlockSpec((H, T, DH), lambda: (0, 0, 0)),
            pl.BlockSpec((H, T, DH), lambda: (0, 0, 0)),
            pl.BlockSpec((D, D), lambda: (0, 0)),
            pl.BlockSpec((D, D), lambda: (0, 0)),
            pl.BlockSpec((D, D), lambda: (0, 0)),
            pl.BlockSpec((1, D), lambda: (0, 0)),
        ],
        out_specs=pl.BlockSpec((T, D), lambda: (0, 0)),
        out_shape=jax.ShapeDtypeStruct((T, D), jnp.float32),
    )(ao, om, Wo, f1, f2, fusion_b.reshape(1, D))

    return out.reshape(B, T, D)


# speedup vs baseline: 1.0427x; 1.0427x over previous
import jax
import jax.numpy as jnp
from jax.experimental import pallas as pl

B, T, D, H, DH, N, TOP_K = 1, 2048, 768, 12, 64, 1024, 8
SCALE = DH ** (-0.5)
NEG = float(jnp.finfo(jnp.float32).min)
T_BLK = 512


def _rope(x, cos, sin):
    half = DH // 2
    rot = jnp.concatenate([-x[:, half:], x[:, :half]], axis=1)
    return x * cos + rot * sin


def _head_kernel(hs_ref, hsb_ref, cos_ref, sin_ref, mem_ref, wq_ref, wk_ref,
                 wv_ref, ao_ref, om_ref):
    hs = hs_ref[...]
    cos = cos_ref[...]
    sin = sin_ref[...]
    q = _rope(jnp.dot(hs, wq_ref[...].T, preferred_element_type=jnp.float32),
              cos, sin)
    qh = q.astype(jnp.bfloat16)
    hsb = hsb_ref[...]
    kh = _rope(jnp.dot(hsb, wk_ref[...].astype(jnp.bfloat16).T,
                       preferred_element_type=jnp.float32),
               cos, sin).astype(jnp.bfloat16)
    v = jnp.dot(hsb, wv_ref[...].astype(jnp.bfloat16).T,
                preferred_element_type=jnp.float32).astype(jnp.bfloat16)
    for c in range(T // T_BLK):
        cols = (c + 1) * T_BLK
        qb = qh[c * T_BLK:(c + 1) * T_BLK, :]
        s = jnp.dot(qb, kh[:cols, :].T,
                    preferred_element_type=jnp.float32) * SCALE
        col = jax.lax.broadcasted_iota(jnp.int32, (T_BLK, cols), 1)
        row = jax.lax.broadcasted_iota(jnp.int32, (T_BLK, cols), 0)
        s = jnp.where(col <= row + c * T_BLK, s, NEG)
        e = jnp.exp(s)
        z = jnp.sum(e, axis=1, keepdims=True)
        o = jnp.dot(e.astype(jnp.bfloat16), v[:cols, :],
                    preferred_element_type=jnp.float32)
        ao_ref[0, c * T_BLK:(c + 1) * T_BLK, :] = (o / z).astype(jnp.bfloat16)
    mem = mem_ref[...]
    k_mem = jnp.dot(mem, wk_ref[...].T, preferred_element_type=jnp.float32)
    v_mem = jnp.dot(mem, wv_ref[...].T, preferred_element_type=jnp.float32)
    sim = jnp.dot(q, k_mem.T, preferred_element_type=jnp.float32) * SCALE
    p1 = jnp.full((T, N // 8), NEG, jnp.float32)
    p2 = p1
    p3 = p1
    for cnk in range(8):
        c = sim[:, cnk * (N // 8):(cnk + 1) * (N // 8)]
        lo = jnp.minimum(p1, c)
        p1 = jnp.maximum(p1, c)
        lo2 = jnp.minimum(p2, lo)
        p2 = jnp.maximum(p2, lo)
        p3 = jnp.maximum(p3, lo2)
    for it in range(TOP_K):
        m = jnp.max(p1, axis=1, keepdims=True)
        if it < TOP_K - 1:
            sel = p1 >= m
            p1 = jnp.where(sel, p2, p1)
            p2 = jnp.where(sel, p3, p2)
            p3 = jnp.where(sel, NEG, p3)
    w = jnp.where(sim >= m, jnp.exp(sim), 0.0)
    z = jnp.sum(w, axis=1, keepdims=True)
    om = jnp.dot(w.astype(jnp.bfloat16), v_mem.astype(jnp.bfloat16),
                 preferred_element_type=jnp.float32)
    om_ref[0] = (om / z).astype(jnp.bfloat16)


def _fusion_kernel(ao_ref, om_ref, wo_ref, f1_ref, f2_ref, b_ref, out_ref):
    ao = jnp.concatenate([ao_ref[h] for h in range(H)], axis=1)
    om = jnp.concatenate([om_ref[h] for h in range(H)], axis=1)
    o_local = jnp.dot(ao, wo_ref[...].astype(jnp.bfloat16).T,
                      preferred_element_type=jnp.float32)
    out = jnp.dot(o_local.astype(jnp.bfloat16),
                  f1_ref[...].astype(jnp.bfloat16).T,
                  preferred_element_type=jnp.float32)
    out += jnp.dot(om, f2_ref[...].astype(jnp.bfloat16).T,
                   preferred_element_type=jnp.float32)
    out_ref[...] = out + b_ref[...]


def kernel(hidden_states, cos, sin, memory, Wq, Wk, Wv, Wo, fusion_W, fusion_b):
    hs = hidden_states[0]
    cs = cos[0]
    sn = sin[0]
    mem = memory[0]
    f1 = fusion_W[:, :D]
    f2 = fusion_W[:, D:]

    head_w = pl.BlockSpec((DH, D), lambda h: (h, 0))
    full2d = lambda a, b: pl.BlockSpec((a, b), lambda h: (0, 0))
    out_head = pl.BlockSpec((1, T, DH), lambda h: (h, 0, 0))

    ao, om = pl.pallas_call(
        _head_kernel,
        grid=(H,),
        in_specs=[full2d(T, D), full2d(T, D), full2d(T, DH), full2d(T, DH),
                  full2d(N, D), head_w, head_w, head_w],
        out_specs=[out_head, out_head],
        out_shape=[jax.ShapeDtypeStruct((H, T, DH), jnp.bfloat16),
                   jax.ShapeDtypeStruct((H, T, DH), jnp.bfloat16)],
    )(hs, hs.astype(jnp.bfloat16), cs, sn, mem, Wq, Wk, Wv)

    out = pl.pallas_call(
        _fusion_kernel,
        in_specs=[
            pl.BlockSpec((H, T, DH), lambda: (0, 0, 0)),
            pl.BlockSpec((H, T, DH), lambda: (0, 0, 0)),
            pl.BlockSpec((D, D), lambda: (0, 0)),
            pl.BlockSpec((D, D), lambda: (0, 0)),
            pl.BlockSpec((D, D), lambda: (0, 0)),
            pl.BlockSpec((1, D), lambda: (0, 0)),
        ],
        out_specs=pl.BlockSpec((T, D), lambda: (0, 0)),
        out_shape=jax.ShapeDtypeStruct((T, D), jnp.float32),
    )(ao, om, Wo, f1, f2, fusion_b.reshape(1, D))

    return out.reshape(B, T, D)
